# Initial kernel scaffold; baseline (speedup 1.0000x reference)
import jax
import jax.numpy as jnp
import numpy as np
from jax.experimental import pallas as pl
from jax.experimental.pallas import tpu as pltpu

L = 8
VOCAB = 4
ED = 64
HID = 128
MP = 2
B = 64
N = 256
EPM = 512
ST = 1.0
TT = 1.5
SR = 0.5
BN = B * N
EB = EPM * B


def _flow_kernel(
    src_ref,
    dst_ref,
    dstr_ref,
    at_ref,
    coords_in_ref,
    ae_ref,
    wie_ref,
    wic_ref,
    wit_ref,
    bi_ref,
    wms_ref,
    wmd_ref,
    w3_ref,
    bm_ref,
    u1_ref,
    u2_ref,
    bu_ref,
    wo1_ref,
    bo1_ref,
    wo2s_ref,
    wo2t_ref,
    bo2s_ref,
    bo2t_ref,
    coords_ref,
    ld_ref,
):
    l = pl.program_id(0)
    f32 = jnp.float32

    @pl.when(l == 0)
    def _():
        coords_ref[...] = coords_in_ref[...]
        ld_ref[...] = jnp.zeros_like(ld_ref)

    lane_n = jax.lax.broadcasted_iota(jnp.int32, (EPM, N), 1)
    g_src = (src_ref[...] == lane_n).astype(f32)
    g_dst = (dst_ref[...] == lane_n).astype(f32)
    sub_n = jax.lax.broadcasted_iota(jnp.int32, (N, EPM), 0)
    s_dst = (sub_n == dstr_ref[...]).astype(f32)

    k3 = (jax.lax.broadcasted_iota(jnp.int32, (3 * B, B), 0) // 3
          == jax.lax.broadcasted_iota(jnp.int32, (3 * B, B), 1)).astype(f32)

    at_oh = (at_ref[...] == jax.lax.broadcasted_iota(jnp.int32, (BN, 8), 1)
             ).astype(f32)

    row_n = jax.lax.broadcasted_iota(jnp.int32, (BN, 1), 0) // B
    active = ((row_n % 2) == (l % 2)).astype(f32)

    coords = coords_ref[...]
    cond = coords * (1.0 - active)

    aew = jnp.dot(ae_ref[0], wie_ref[0])
    aew8 = jnp.concatenate([aew, jnp.zeros((4, HID), f32)], axis=0)
    tconst = ST * wit_ref[0, 0, :] + TT * wit_ref[0, 1, :]
    h = jnp.dot(at_oh, aew8) + jnp.dot(cond, wic_ref[0]) \
        + tconst[None, :] + bi_ref[...]
    h = jnp.maximum(h, 0.0)

    cond_b = cond.reshape(N, 3 * B)
    d = jnp.dot(g_src, cond_b) - jnp.dot(g_dst, cond_b)
    s2 = jnp.dot(d * d, k3)
    dist = jnp.sqrt(s2 + 1e-8).reshape(EB, 1)

    for m in range(MP):
        a1 = jnp.dot(h, wms_ref[0, m])
        a2 = jnp.dot(h, wmd_ref[0, m])
        eb = jnp.dot(g_src, a1.reshape(N, B * HID)) \
            + jnp.dot(g_dst, a2.reshape(N, B * HID))
        msg = eb.reshape(EB, HID) + dist * w3_ref[0, m] + bm_ref[0, m][None, :]
        msg = jnp.maximum(msg, 0.0)
        agg = jnp.dot(s_dst, msg.reshape(EPM, B * HID))
        h = jnp.dot(h, u1_ref[0, m]) \
            + jnp.dot(agg.reshape(BN, HID), u2_ref[0, m]) \
            + bu_ref[0, m][None, :]
        h = jnp.maximum(h, 0.0)

    h1 = jnp.maximum(jnp.dot(h, wo1_ref[0]) + bo1_ref[...], 0.0)
    raw_s = jnp.dot(h1, wo2s_ref[0]) + bo2s_ref[0]
    raw_t = jnp.dot(h1, wo2t_ref[0]) + bo2t_ref[0]
    scale = SR * jnp.tanh(raw_s) * active
    shift = raw_t * active
    coords_ref[...] = coords * jnp.exp(scale) + shift
    ld_ref[...] += jnp.sum(scale.reshape(N, 3 * B), axis=0, keepdims=True)


def kernel(coordinates, atom_types, adj_list, atom_embed, W_in, b_in,
           W_msg, b_msg, W_upd, b_upd, W_o1, b_o1, W_o2, b_o2):
    f32 = jnp.float32
    coords_nm = coordinates.transpose(1, 0, 2).reshape(BN, 3).astype(f32)
    at_nm = atom_types.transpose(1, 0).reshape(BN, 1).astype(jnp.int32)
    src = adj_list[:, 0:1].astype(jnp.int32)
    dst = adj_list[:, 1:2].astype(jnp.int32)
    dstr = adj_list[:, 1][None, :].astype(jnp.int32)

    wie = W_in[:, :ED, :]
    wic = W_in[:, ED:ED + 3, :]
    wit = W_in[:, ED + 3:ED + 5, :]
    wms = W_msg[:, :, :HID, :]
    wmd = W_msg[:, :, HID:2 * HID, :]
    w3 = W_msg[:, :, 2 * HID:2 * HID + 1, :]
    u1 = W_upd[:, :, :HID, :]
    u2 = W_upd[:, :, HID:, :]
    wo2s = W_o2[:, :, :3]
    wo2t = W_o2[:, :, 3:]
    bo2s = b_o2[:, None, :3]
    bo2t = b_o2[:, None, 3:]

    def cm(*shape):
        return pl.BlockSpec(shape, lambda l, _n=len(shape): (0,) * _n)

    def lm(*shape):
        return pl.BlockSpec((1,) + shape, lambda l, _n=len(shape): (l,) + (0,) * _n)

    coords_out, ld = pl.pallas_call(
        _flow_kernel,
        grid=(L,),
        in_specs=[
            cm(EPM, 1), cm(EPM, 1), cm(1, EPM), cm(BN, 1), cm(BN, 3),
            lm(VOCAB, ED), lm(ED, HID), lm(3, HID), lm(2, HID), lm(HID),
            lm(MP, HID, HID), lm(MP, HID, HID), lm(MP, 1, HID), lm(MP, HID),
            lm(MP, HID, HID), lm(MP, HID, HID), lm(MP, HID),
            lm(HID, HID), lm(HID), lm(HID, 3), lm(HID, 3), lm(1, 3), lm(1, 3),
        ],
        out_specs=[
            pl.BlockSpec((BN, 3), lambda l: (0, 0)),
            pl.BlockSpec((1, 3 * B), lambda l: (0, 0)),
        ],
        out_shape=[
            jax.ShapeDtypeStruct((BN, 3), f32),
            jax.ShapeDtypeStruct((1, 3 * B), f32),
        ],
    )(src, dst, dstr, at_nm, coords_nm, atom_embed, wie, wic, wit, b_in,
      wms, wmd, w3, b_msg, u1, u2, b_upd, W_o1, b_o1, wo2s, wo2t, bo2s, bo2t)

    coords_final = coords_out.reshape(N, B, 3).transpose(1, 0, 2)
    log_det = ld.reshape(B, 3).sum(axis=-1)
    return coords_final, log_det


# speedup vs baseline: 14.3930x; 14.3930x over previous
import jax
import jax.numpy as jnp
import numpy as np
from jax.experimental import pallas as pl
from jax.experimental.pallas import tpu as pltpu

L = 8
VOCAB = 4
ED = 64
HID = 128
MP = 2
B = 64
N = 256
EPM = 512
ST = 1.0
TT = 1.5
SR = 0.5
NH = 4
BH = B // NH
BNH = N * BH
EBH = EPM * BH
BN = B * N


def _flow_kernel(
    src_ref,
    dst_ref,
    dstr_ref,
    at_ref,
    coords_in_ref,
    ae_ref,
    wie_ref,
    wic_ref,
    wit_ref,
    bi_ref,
    wms_ref,
    wmd_ref,
    w3_ref,
    bm_ref,
    u1_ref,
    u2_ref,
    bu_ref,
    wo1_ref,
    bo1_ref,
    wo2s_ref,
    wo2t_ref,
    bo2s_ref,
    bo2t_ref,
    coords_out_ref,
    ld_out_ref,
    coords_s,
    ld_s,
):
    l = pl.program_id(0)
    g = pl.program_id(1)
    f32 = jnp.float32
    rows = pl.ds(g * BNH, BNH)
    brows = pl.ds(g * BH, BH)

    @pl.when(l == 0)
    def _():
        coords_s[rows, :] = coords_in_ref[...]
        ld_s[brows, :] = jnp.zeros((BH, 3), f32)

    lane_n = jax.lax.broadcasted_iota(jnp.int32, (EPM, N), 1)
    g_src = (src_ref[...] == lane_n).astype(f32)
    g_dst = (dst_ref[...] == lane_n).astype(f32)
    sub_n = jax.lax.broadcasted_iota(jnp.int32, (N, EPM), 0)
    s_dst = (sub_n == dstr_ref[...]).astype(f32)

    sel3 = (jax.lax.broadcasted_iota(jnp.int32, (3, HID), 0)
            == jax.lax.broadcasted_iota(jnp.int32, (3, HID), 1)).astype(f32)
    m_b = (jax.lax.broadcasted_iota(jnp.int32, (BH, BNH), 1) % BH
           == jax.lax.broadcasted_iota(jnp.int32, (BH, BNH), 0)).astype(f32)

    at_oh = (at_ref[...] == jax.lax.broadcasted_iota(jnp.int32, (BNH, 8), 1)
             ).astype(f32)

    row_n = jax.lax.broadcasted_iota(jnp.int32, (BNH, 1), 0) // BH
    active = ((row_n % 2) == (l % 2)).astype(f32)

    coords = coords_s[rows, :]
    cond = coords * (1.0 - active)

    aew = jnp.dot(ae_ref[0], wie_ref[0])
    aew8 = jnp.concatenate([aew, jnp.zeros((4, HID), f32)], axis=0)
    tconst = ST * wit_ref[0, 0, :] + TT * wit_ref[0, 1, :]
    h = jnp.dot(at_oh, aew8) + jnp.dot(cond, wic_ref[0]) \
        + tconst[None, :] + bi_ref[0]
    h = jnp.maximum(h, 0.0)

    cond_pad = jnp.dot(cond, sel3)
    dpad = jnp.dot(g_src - g_dst, cond_pad.reshape(N, BH * HID))
    d2 = (dpad * dpad).reshape(EBH, HID)
    s2 = jnp.sum(d2[:, :3], axis=1, keepdims=True)
    dist = jnp.sqrt(s2 + 1e-8)

    for m in range(MP):
        a1 = jnp.dot(h, wms_ref[0, m])
        a2 = jnp.dot(h, wmd_ref[0, m])
        eb = jnp.dot(g_src, a1.reshape(N, BH * HID)) \
            + jnp.dot(g_dst, a2.reshape(N, BH * HID))
        msg = eb.reshape(EBH, HID) + dist * w3_ref[0, m] + bm_ref[0, m][None, :]
        msg = jnp.maximum(msg, 0.0)
        agg = jnp.dot(s_dst, msg.reshape(EPM, BH * HID))
        h = jnp.dot(h, u1_ref[0, m]) \
            + jnp.dot(agg.reshape(BNH, HID), u2_ref[0, m]) \
            + bu_ref[0, m][None, :]
        h = jnp.maximum(h, 0.0)

    h1 = jnp.maximum(jnp.dot(h, wo1_ref[0]) + bo1_ref[0], 0.0)
    raw_s = jnp.dot(h1, wo2s_ref[0]) + bo2s_ref[0]
    raw_t = jnp.dot(h1, wo2t_ref[0]) + bo2t_ref[0]
    scale = SR * jnp.tanh(raw_s) * active
    shift = raw_t * active
    new_coords = coords * jnp.exp(scale) + shift
    coords_s[rows, :] = new_coords
    coords_out_ref[0] = new_coords
    ldg = ld_s[brows, :] + jnp.dot(m_b, scale, precision=jax.lax.Precision.HIGHEST)
    ld_s[brows, :] = ldg
    ld_out_ref[0] = ldg


def kernel(coordinates, atom_types, adj_list, atom_embed, W_in, b_in,
           W_msg, b_msg, W_upd, b_upd, W_o1, b_o1, W_o2, b_o2):
    f32 = jnp.float32
    coords_nm = coordinates.reshape(NH, BH, N, 3).transpose(0, 2, 1, 3) \
        .reshape(BN, 3).astype(f32)
    at_nm = atom_types.reshape(NH, BH, N).transpose(0, 2, 1) \
        .reshape(BN, 1).astype(jnp.int32)
    src = adj_list[:, 0:1].astype(jnp.int32)
    dst = adj_list[:, 1:2].astype(jnp.int32)
    dstr = adj_list[:, 1][None, :].astype(jnp.int32)

    wie = W_in[:, :ED, :]
    wic = W_in[:, ED:ED + 3, :]
    wit = W_in[:, ED + 3:ED + 5, :]
    wms = W_msg[:, :, :HID, :]
    wmd = W_msg[:, :, HID:2 * HID, :]
    w3 = W_msg[:, :, 2 * HID:2 * HID + 1, :]
    u1 = W_upd[:, :, :HID, :]
    u2 = W_upd[:, :, HID:, :]
    b_in3 = b_in[:, None, :]
    b_o13 = b_o1[:, None, :]
    wo2s = W_o2[:, :, :3]
    wo2t = W_o2[:, :, 3:]
    bo2s = b_o2[:, None, :3]
    bo2t = b_o2[:, None, 3:]

    def cm(*shape):
        return pl.BlockSpec(shape, lambda l, g, _n=len(shape): (0,) * _n)

    def gm(*shape):
        return pl.BlockSpec(shape, lambda l, g, _n=len(shape): (g,) + (0,) * (_n - 1))

    def lm(*shape):
        return pl.BlockSpec((1,) + shape,
                            lambda l, g, _n=len(shape): (l,) + (0,) * _n)

    coords_out, ld = pl.pallas_call(
        _flow_kernel,
        grid=(L, NH),
        in_specs=[
            cm(EPM, 1), cm(EPM, 1), cm(1, EPM), gm(BNH, 1), gm(BNH, 3),
            lm(VOCAB, ED), lm(ED, HID), lm(3, HID), lm(2, HID), lm(1, HID),
            lm(MP, HID, HID), lm(MP, HID, HID), lm(MP, 1, HID), lm(MP, HID),
            lm(MP, HID, HID), lm(MP, HID, HID), lm(MP, HID),
            lm(HID, HID), lm(1, HID), lm(HID, 3), lm(HID, 3), lm(1, 3), lm(1, 3),
        ],
        out_specs=[
            pl.BlockSpec((1, BNH, 3), lambda l, g: (l, g, 0)),
            pl.BlockSpec((1, BH, 3), lambda l, g: (l, g, 0)),
        ],
        out_shape=[
            jax.ShapeDtypeStruct((L, BN, 3), f32),
            jax.ShapeDtypeStruct((L, B, 3), f32),
        ],
        scratch_shapes=[
            pltpu.VMEM((BN, 3), f32),
            pltpu.VMEM((B, 3), f32),
        ],
    )(src, dst, dstr, at_nm, coords_nm, atom_embed, wie, wic, wit, b_in3,
      wms, wmd, w3, b_msg, u1, u2, b_upd, W_o1, b_o13, wo2s, wo2t, bo2s, bo2t)

    coords_final = coords_out[L - 1].reshape(NH, N, BH, 3).transpose(0, 2, 1, 3) \
        .reshape(B, N, 3)
    log_det = ld[L - 1].sum(axis=-1)
    return coords_final, log_det


# speedup vs baseline: 14.8224x; 1.0298x over previous
import jax
import jax.numpy as jnp
import numpy as np
from jax.experimental import pallas as pl
from jax.experimental.pallas import tpu as pltpu

L = 8
VOCAB = 4
ED = 64
HID = 128
MP = 2
B = 64
N = 256
EPM = 512
ST = 1.0
TT = 1.5
SR = 0.5
NH = 4
BH = B // NH
BNH = N * BH
EBH = EPM * BH
BN = B * N


def _flow_kernel(
    src_ref,
    dst_ref,
    dstr_ref,
    at_ref,
    coords_in_ref,
    ae_ref,
    wie_ref,
    wic_ref,
    wit_ref,
    bi_ref,
    wms_ref,
    wmd_ref,
    w3_ref,
    bm_ref,
    wu_ref,
    bu_ref,
    wo1_ref,
    bo1_ref,
    wo2s_ref,
    wo2t_ref,
    bo2s_ref,
    bo2t_ref,
    coords_out_ref,
    ld_out_ref,
    coords_s,
    ld_s,
    gs_s,
    gd_s,
    sd_s,
    oh_s,
):
    l = pl.program_id(0)
    g = pl.program_id(1)
    f32 = jnp.float32
    rows = pl.ds(g * BNH, BNH)
    brows = pl.ds(g * BH, BH)

    @pl.when(l == 0)
    def _():
        coords_s[rows, :] = coords_in_ref[...]
        ld_s[brows, :] = jnp.zeros((BH, 1), f32)
        oh_s[rows, :] = (at_ref[...] == jax.lax.broadcasted_iota(
            jnp.int32, (BNH, 8), 1)).astype(f32)

    @pl.when(jnp.logical_and(l == 0, g == 0))
    def _():
        lane_n = jax.lax.broadcasted_iota(jnp.int32, (EPM, N), 1)
        gs_s[...] = (src_ref[...] == lane_n).astype(f32)
        gd_s[...] = (dst_ref[...] == lane_n).astype(f32)
        sub_n = jax.lax.broadcasted_iota(jnp.int32, (N, EPM), 0)
        sd_s[...] = (sub_n == dstr_ref[...]).astype(f32)

    g_src = gs_s[...]
    g_dst = gd_s[...]
    s_dst = sd_s[...]
    at_oh = oh_s[rows, :]

    sel3 = (jax.lax.broadcasted_iota(jnp.int32, (3, HID), 0)
            == jax.lax.broadcasted_iota(jnp.int32, (3, HID), 1)).astype(f32)

    row_n = jax.lax.broadcasted_iota(jnp.int32, (BNH, 1), 0) // BH
    active = ((row_n % 2) == (l % 2)).astype(f32)

    coords = coords_s[rows, :]
    cond = coords * (1.0 - active)

    aew = jnp.dot(ae_ref[0], wie_ref[0])
    aew8 = jnp.concatenate([aew, jnp.zeros((4, HID), f32)], axis=0)
    tconst = ST * wit_ref[0, 0, :] + TT * wit_ref[0, 1, :]
    h = jnp.dot(at_oh, aew8) + jnp.dot(cond, wic_ref[0]) \
        + tconst[None, :] + bi_ref[0]
    h = jnp.maximum(h, 0.0)

    cond_pad = jnp.dot(cond, sel3)
    dpad = jnp.dot(g_src - g_dst, cond_pad.reshape(N, BH * HID))
    d2 = (dpad * dpad).reshape(EBH, HID)
    s2 = jnp.sum(d2[:, :3], axis=1, keepdims=True)
    dist = jnp.sqrt(s2 + 1e-8)

    for m in range(MP):
        a1 = jnp.dot(h, wms_ref[0, m])
        a2 = jnp.dot(h, wmd_ref[0, m])
        eb = jnp.dot(g_src, a1.reshape(N, BH * HID)) \
            + jnp.dot(g_dst, a2.reshape(N, BH * HID))
        msg = eb.reshape(EBH, HID) + dist * w3_ref[0, m] + bm_ref[0, m][None, :]
        msg = jnp.maximum(msg, 0.0)
        agg = jnp.dot(s_dst, msg.reshape(EPM, BH * HID))
        hcat = jnp.concatenate([h, agg.reshape(BNH, HID)], axis=1)
        h = jnp.maximum(jnp.dot(hcat, wu_ref[0, m]) + bu_ref[0, m][None, :], 0.0)

    h1 = jnp.maximum(jnp.dot(h, wo1_ref[0]) + bo1_ref[0], 0.0)
    raw_s = jnp.dot(h1, wo2s_ref[0]) + bo2s_ref[0]
    raw_t = jnp.dot(h1, wo2t_ref[0]) + bo2t_ref[0]
    scale = SR * jnp.tanh(raw_s) * active
    shift = raw_t * active
    new_coords = coords * jnp.exp(scale) + shift
    coords_s[rows, :] = new_coords
    coords_out_ref[0] = new_coords
    m_b = (jax.lax.broadcasted_iota(jnp.int32, (BH, BNH), 1) % BH
           == jax.lax.broadcasted_iota(jnp.int32, (BH, BNH), 0)).astype(f32)
    scale_sum = jnp.sum(scale, axis=1, keepdims=True)
    ldg = ld_s[brows, :] + jnp.dot(m_b, scale_sum,
                                   precision=jax.lax.Precision.HIGHEST)
    ld_s[brows, :] = ldg
    ld_out_ref[0] = ldg


def kernel(coordinates, atom_types, adj_list, atom_embed, W_in, b_in,
           W_msg, b_msg, W_upd, b_upd, W_o1, b_o1, W_o2, b_o2):
    f32 = jnp.float32
    coords_nm = coordinates.reshape(NH, BH, N, 3).transpose(0, 2, 1, 3) \
        .reshape(BN, 3).astype(f32)
    at_nm = atom_types.reshape(NH, BH, N).transpose(0, 2, 1) \
        .reshape(BN, 1).astype(jnp.int32)
    src = adj_list[:, 0:1].astype(jnp.int32)
    dst = adj_list[:, 1:2].astype(jnp.int32)
    dstr = adj_list[:, 1][None, :].astype(jnp.int32)

    wie = W_in[:, :ED, :]
    wic = W_in[:, ED:ED + 3, :]
    wit = W_in[:, ED + 3:ED + 5, :]
    wms = W_msg[:, :, :HID, :]
    wmd = W_msg[:, :, HID:2 * HID, :]
    w3 = W_msg[:, :, 2 * HID:2 * HID + 1, :]
    b_in3 = b_in[:, None, :]
    b_o13 = b_o1[:, None, :]
    wo2s = W_o2[:, :, :3]
    wo2t = W_o2[:, :, 3:]
    bo2s = b_o2[:, None, :3]
    bo2t = b_o2[:, None, 3:]

    def cm(*shape):
        return pl.BlockSpec(shape, lambda l, g, _n=len(shape): (0,) * _n)

    def gm(*shape):
        return pl.BlockSpec(shape, lambda l, g, _n=len(shape): (g,) + (0,) * (_n - 1))

    def lm(*shape):
        return pl.BlockSpec((1,) + shape,
                            lambda l, g, _n=len(shape): (l,) + (0,) * _n)

    coords_out, ld = pl.pallas_call(
        _flow_kernel,
        grid=(L, NH),
        in_specs=[
            cm(EPM, 1), cm(EPM, 1), cm(1, EPM), gm(BNH, 1), gm(BNH, 3),
            lm(VOCAB, ED), lm(ED, HID), lm(3, HID), lm(2, HID), lm(1, HID),
            lm(MP, HID, HID), lm(MP, HID, HID), lm(MP, 1, HID), lm(MP, HID),
            lm(MP, 2 * HID, HID), lm(MP, HID),
            lm(HID, HID), lm(1, HID), lm(HID, 3), lm(HID, 3), lm(1, 3), lm(1, 3),
        ],
        out_specs=[
            pl.BlockSpec((1, BNH, 3), lambda l, g: (l, g, 0)),
            pl.BlockSpec((1, BH, 1), lambda l, g: (l, g, 0)),
        ],
        out_shape=[
            jax.ShapeDtypeStruct((L, BN, 3), f32),
            jax.ShapeDtypeStruct((L, B, 1), f32),
        ],
        scratch_shapes=[
            pltpu.VMEM((BN, 3), f32),
            pltpu.VMEM((B, 1), f32),
            pltpu.VMEM((EPM, N), f32),
            pltpu.VMEM((EPM, N), f32),
            pltpu.VMEM((N, EPM), f32),
            pltpu.VMEM((BN, 8), f32),
        ],
    )(src, dst, dstr, at_nm, coords_nm, atom_embed, wie, wic, wit, b_in3,
      wms, wmd, w3, b_msg, W_upd, b_upd, W_o1, b_o13, wo2s, wo2t, bo2s, bo2t)

    coords_final = coords_out[L - 1].reshape(NH, N, BH, 3).transpose(0, 2, 1, 3) \
        .reshape(B, N, 3)
    log_det = ld[L - 1, :, 0]
    return coords_final, log_det


# speedup vs baseline: 17.2330x; 1.1626x over previous
import jax
import jax.numpy as jnp
import numpy as np
from jax.experimental import pallas as pl
from jax.experimental.pallas import tpu as pltpu

L = 8
VOCAB = 4
ED = 64
HID = 128
MP = 2
B = 64
N = 256
EPM = 512
ST = 1.0
TT = 1.5
SR = 0.5
NH = 4
BH = B // NH
BNH = N * BH
EBH = EPM * BH
BN = B * N


def _flow_kernel(
    src_ref,
    dst_ref,
    dstr_ref,
    at_ref,
    coords_in_ref,
    ae_ref,
    wie_ref,
    wic_ref,
    wit_ref,
    bi_ref,
    wms_ref,
    wmd_ref,
    w3_ref,
    bm_ref,
    wu_ref,
    bu_ref,
    wo1_ref,
    bo1_ref,
    wo2s_ref,
    wo2t_ref,
    bo2s_ref,
    bo2t_ref,
    coords_out_ref,
    ld_out_ref,
    coords_s,
    ld_s,
    gs_s,
    gd_s,
    gdf_s,
    sd_s,
    oh_s,
    k2_s,
):
    l = pl.program_id(0)
    g = pl.program_id(1)
    f32 = jnp.float32
    rows = pl.ds(g * BNH, BNH)
    brows = pl.ds(g * BH, BH)

    @pl.when(l == 0)
    def _():
        coords_s[rows, :] = coords_in_ref[...]
        ld_s[brows, :] = jnp.zeros((BH, 1), f32)
        oh_s[rows, :] = (at_ref[...] == jax.lax.broadcasted_iota(
            jnp.int32, (BNH, 8), 1)).astype(f32)

    @pl.when(jnp.logical_and(l == 0, g == 0))
    def _():
        lane_n = jax.lax.broadcasted_iota(jnp.int32, (EPM, N), 1)
        gs = (src_ref[...] == lane_n).astype(f32)
        gd = (dst_ref[...] == lane_n).astype(f32)
        gs_s[...] = gs
        gd_s[...] = gd
        gdf_s[...] = gs - gd
        sub_n = jax.lax.broadcasted_iota(jnp.int32, (N, EPM), 0)
        sd_s[...] = (sub_n == dstr_ref[...]).astype(f32)
        jj = jax.lax.broadcasted_iota(jnp.int32, (BH * HID, BH), 0)
        bb = jax.lax.broadcasted_iota(jnp.int32, (BH * HID, BH), 1)
        k2_s[...] = ((jj // HID == bb) & (jj % HID < 3)).astype(f32)

    g_src = gs_s[...]
    g_dst = gd_s[...]
    s_dst = sd_s[...]
    at_oh = oh_s[rows, :]

    sel3 = (jax.lax.broadcasted_iota(jnp.int32, (3, HID), 0)
            == jax.lax.broadcasted_iota(jnp.int32, (3, HID), 1)).astype(f32)
    m16 = (jax.lax.broadcasted_iota(jnp.int32, (BH, BH * HID), 1) // HID
           == jax.lax.broadcasted_iota(jnp.int32, (BH, BH * HID), 0)).astype(f32)

    row_n = jax.lax.broadcasted_iota(jnp.int32, (BNH, 1), 0) // BH
    active = ((row_n % 2) == (l % 2)).astype(f32)

    coords = coords_s[rows, :]
    cond = coords * (1.0 - active)

    aew = jnp.dot(ae_ref[0], wie_ref[0])
    aew8 = jnp.concatenate([aew, jnp.zeros((4, HID), f32)], axis=0)
    tconst = ST * wit_ref[0, 0, :] + TT * wit_ref[0, 1, :]
    wcat = jnp.concatenate([wic_ref[0], sel3], axis=1)
    ccat = jnp.dot(cond, wcat)
    h = jnp.dot(at_oh, aew8) + ccat[:, :HID] \
        + tconst[None, :] + bi_ref[0]
    h = jnp.maximum(h, 0.0)

    cpb = ccat[:, HID:].reshape(N, BH * HID)
    dpad = jnp.dot(gdf_s[...], cpb)
    s2b = jnp.dot(dpad * dpad, k2_s[...])
    distb = jnp.sqrt(s2b + 1e-8)
    ones_col = jnp.ones((EPM, 1), f32)
    distb1 = jnp.concatenate([distb, ones_col], axis=1)

    for m in range(MP):
        a1 = jnp.dot(h, wms_ref[0, m])
        a2 = jnp.dot(h, wmd_ref[0, m])
        eb = jnp.dot(g_src, a1.reshape(N, BH * HID)) \
            + jnp.dot(g_dst, a2.reshape(N, BH * HID))
        w3t = m16 * jnp.broadcast_to(w3_ref[0, m], (BH, HID)).reshape(1, BH * HID)
        bmt = jnp.broadcast_to(bm_ref[0, m][None, :], (BH, HID)).reshape(1, BH * HID)
        dterm = jnp.dot(distb1, jnp.concatenate([w3t, bmt], axis=0))
        msg = jnp.maximum(eb + dterm, 0.0)
        agg = jnp.dot(s_dst, msg)
        hcat = jnp.concatenate([h, agg.reshape(BNH, HID)], axis=1)
        h = jnp.maximum(jnp.dot(hcat, wu_ref[0, m]) + bu_ref[0, m][None, :], 0.0)

    h1 = jnp.maximum(jnp.dot(h, wo1_ref[0]) + bo1_ref[0], 0.0)
    raw_s = jnp.dot(h1, wo2s_ref[0]) + bo2s_ref[0]
    raw_t = jnp.dot(h1, wo2t_ref[0]) + bo2t_ref[0]
    scale = SR * jnp.tanh(raw_s) * active
    shift = raw_t * active
    new_coords = coords * jnp.exp(scale) + shift
    coords_s[rows, :] = new_coords
    coords_out_ref[0] = new_coords
    m_b = (jax.lax.broadcasted_iota(jnp.int32, (BH, BNH), 1) % BH
           == jax.lax.broadcasted_iota(jnp.int32, (BH, BNH), 0)).astype(f32)
    scale_sum = jnp.sum(scale, axis=1, keepdims=True)
    ldg = ld_s[brows, :] + jnp.dot(m_b, scale_sum,
                                   precision=jax.lax.Precision.HIGHEST)
    ld_s[brows, :] = ldg
    ld_out_ref[0] = ldg


def kernel(coordinates, atom_types, adj_list, atom_embed, W_in, b_in,
           W_msg, b_msg, W_upd, b_upd, W_o1, b_o1, W_o2, b_o2):
    f32 = jnp.float32
    coords_nm = coordinates.reshape(NH, BH, N, 3).transpose(0, 2, 1, 3) \
        .reshape(BN, 3).astype(f32)
    at_nm = atom_types.reshape(NH, BH, N).transpose(0, 2, 1) \
        .reshape(BN, 1).astype(jnp.int32)
    src = adj_list[:, 0:1].astype(jnp.int32)
    dst = adj_list[:, 1:2].astype(jnp.int32)
    dstr = adj_list[:, 1][None, :].astype(jnp.int32)

    wie = W_in[:, :ED, :]
    wic = W_in[:, ED:ED + 3, :]
    wit = W_in[:, ED + 3:ED + 5, :]
    wms = W_msg[:, :, :HID, :]
    wmd = W_msg[:, :, HID:2 * HID, :]
    w3 = W_msg[:, :, 2 * HID:2 * HID + 1, :]
    b_in3 = b_in[:, None, :]
    b_o13 = b_o1[:, None, :]
    wo2s = W_o2[:, :, :3]
    wo2t = W_o2[:, :, 3:]
    bo2s = b_o2[:, None, :3]
    bo2t = b_o2[:, None, 3:]

    def cm(*shape):
        return pl.BlockSpec(shape, lambda l, g, _n=len(shape): (0,) * _n)

    def gm(*shape):
        return pl.BlockSpec(shape, lambda l, g, _n=len(shape): (g,) + (0,) * (_n - 1))

    def lm(*shape):
        return pl.BlockSpec((1,) + shape,
                            lambda l, g, _n=len(shape): (l,) + (0,) * _n)

    coords_out, ld = pl.pallas_call(
        _flow_kernel,
        grid=(L, NH),
        in_specs=[
            cm(EPM, 1), cm(EPM, 1), cm(1, EPM), gm(BNH, 1), gm(BNH, 3),
            lm(VOCAB, ED), lm(ED, HID), lm(3, HID), lm(2, HID), lm(1, HID),
            lm(MP, HID, HID), lm(MP, HID, HID), lm(MP, 1, HID), lm(MP, HID),
            lm(MP, 2 * HID, HID), lm(MP, HID),
            lm(HID, HID), lm(1, HID), lm(HID, 3), lm(HID, 3), lm(1, 3), lm(1, 3),
        ],
        out_specs=[
            pl.BlockSpec((1, BNH, 3), lambda l, g: (l, g, 0)),
            pl.BlockSpec((1, BH, 1), lambda l, g: (l, g, 0)),
        ],
        out_shape=[
            jax.ShapeDtypeStruct((L, BN, 3), f32),
            jax.ShapeDtypeStruct((L, B, 1), f32),
        ],
        scratch_shapes=[
            pltpu.VMEM((BN, 3), f32),
            pltpu.VMEM((B, 1), f32),
            pltpu.VMEM((EPM, N), f32),
            pltpu.VMEM((EPM, N), f32),
            pltpu.VMEM((EPM, N), f32),
            pltpu.VMEM((N, EPM), f32),
            pltpu.VMEM((BN, 8), f32),
            pltpu.VMEM((BH * HID, BH), f32),
        ],
    )(src, dst, dstr, at_nm, coords_nm, atom_embed, wie, wic, wit, b_in3,
      wms, wmd, w3, b_msg, W_upd, b_upd, W_o1, b_o13, wo2s, wo2t, bo2s, bo2t)

    coords_final = coords_out[L - 1].reshape(NH, N, BH, 3).transpose(0, 2, 1, 3) \
        .reshape(B, N, 3)
    log_det = ld[L - 1, :, 0]
    return coords_final, log_det
